# Initial kernel scaffold; baseline (speedup 1.0000x reference)
import jax
import jax.numpy as jnp
from jax.experimental import pallas as pl


def kernel(x, edge_index, W, b):
    raise NotImplementedError("write your pallas kernel here")


# speedup vs baseline: 29.9687x; 29.9687x over previous
import functools

import jax
import jax.numpy as jnp
from jax import lax
from jax.experimental import pallas as pl
from jax.experimental.pallas import tpu as pltpu
from jax.experimental.pallas import tpu_sc as plsc

NC = 2
NS = 16
NW = NC * NS
CH = 125
KB = 16
DLANES = 128


def _deg_body(npad, n_vecs, row16, zeros_hb, deg_out,
              row_v, hist2d, iota_v, acc_sh):
    c = lax.axis_index("c")
    s = lax.axis_index("s")
    wid = c * NS + s
    nrow = npad // DLANES

    @pl.when(s < nrow // 8)
    def _():
        pltpu.sync_copy(zeros_hb.at[pl.ds(s * 8, 8)],
                        acc_sh.at[pl.ds(s * 8, 8)])

    pltpu.sync_copy(row16.at[wid], row_v)

    @pl.loop(0, nrow)
    def _(r):
        @pl.loop(0, DLANES // 16)
        def _(q):
            hist2d[r, pl.ds(q * 16, 16)] = jnp.zeros((16,), jnp.float32)

    @pl.loop(0, nrow // 16)
    def _(q):
        iota_v[pl.ds(q * 16, 16)] = lax.iota(jnp.int32, 16) + q * 16

    ones = jnp.ones((16,), jnp.float32)

    @pl.loop(0, n_vecs)
    def _(j):
        idx = row_v[j, :]
        hi = lax.shift_right_logical(idx, 7)
        lo = lax.bitwise_and(idx, 127)
        plsc.addupdate_scatter(hist2d, [hi, lo], ones)

    plsc.subcore_barrier()
    pltpu.sync_copy(hist2d, acc_sh.at[iota_v], add=True)
    plsc.subcore_barrier()

    @pl.when(s < nrow // 8)
    def _():
        pltpu.sync_copy(acc_sh.at[pl.ds(s * 8, 8)],
                        deg_out.at[c, pl.ds(s * 8, 8)])


def _agg_body(npad, n_chunks, d_in, row2d, col2d, y, zeros, z_out,
              row_v, col_v, g0, g1, z_sh, gsem0, gsem1, ssem0, ssem1):
    c = lax.axis_index("c")
    s = lax.axis_index("s")
    wid = c * NS + s
    npt = npad // NS
    pltpu.sync_copy(zeros.at[pl.ds(s * npt, npt)],
                    z_sh.at[pl.ds(s * npt, npt)])
    plsc.subcore_barrier()

    def wait_gather(buf, sem):
        pltpu.make_async_copy(y.at[col_v.at[0]], buf, sem).wait()

    def wait_scatter(buf, sem):
        pltpu.make_async_copy(buf, z_sh.at[row_v.at[0]], sem).wait()

    def step(j, cur, curg, curs, oth, othg, oths):
        wait_gather(cur, curg)
        pltpu.async_copy(cur, z_sh.at[row_v.at[j]], curs, add=True)

        @pl.when(j + 1 < KB)
        def _():
            @pl.when(j >= 1)
            def _():
                wait_scatter(oth, oths)

            pltpu.async_copy(y.at[col_v.at[j + 1]], oth, othg)

    @pl.loop(0, n_chunks, step=KB)
    def _(jo):
        pltpu.sync_copy(row2d.at[wid, pl.ds(jo, KB)], row_v)
        pltpu.sync_copy(col2d.at[wid, pl.ds(jo, KB)], col_v)
        pltpu.async_copy(y.at[col_v.at[0]], g0, gsem0)

        @pl.loop(0, KB)
        def _(j):
            @pl.when(j % 2 == 0)
            def _():
                step(j, g0, gsem0, ssem0, g1, gsem1, ssem1)

            @pl.when(j % 2 == 1)
            def _():
                step(j, g1, gsem1, ssem1, g0, gsem0, ssem0)

        wait_scatter(g0, ssem0)
        wait_scatter(g1, ssem1)

    plsc.subcore_barrier()
    pltpu.sync_copy(z_sh.at[pl.ds(s * npt, npt)],
                    z_out.at[c, pl.ds(s * npt, npt)])


def _dis_from_deg(dblk):
    deg = dblk[0] + dblk[1]
    return jnp.where(deg > 0.0, lax.rsqrt(deg), 0.0)


def _scale_body(x_ref, deg_ref, y_ref):
    dis = _dis_from_deg(deg_ref[...])
    y_ref[...] = x_ref[...] * dis[:, None]


def _out_body(z_ref, deg_ref, w_ref, b_ref, o_ref):
    z = z_ref[...]
    dis = _dis_from_deg(deg_ref[...])
    agg = (z[0] + z[1]) * dis[:, None]
    acc = lax.dot_general(agg, w_ref[...], (((1,), (1,)), ((), ())),
                          preferred_element_type=jnp.float32)
    o_ref[...] = jnp.maximum(acc + b_ref[...], 0.0)


def kernel(x, edge_index, W, b):
    n, d_in = x.shape
    d_out = W.shape[0]
    e = edge_index.shape[1]
    assert e % (NW * CH) == 0
    ept = e // NW
    n_chunks = ept // CH
    npad = ((n + 1023) // 1024) * 1024

    row2d = edge_index[0].reshape(NW, n_chunks, CH)
    col2d = edge_index[1].reshape(NW, n_chunks, CH)
    n_vecs = ept // 16
    row16 = edge_index[0].reshape(NW, n_vecs, 16)
    nrow = npad // DLANES
    zeros_row = jnp.zeros((nrow, DLANES), jnp.float32)
    zeros_nd = jnp.zeros((npad, d_in), jnp.float32)

    mesh = plsc.VectorSubcoreMesh(core_axis_name="c", subcore_axis_name="s")

    deg = pl.kernel(
        functools.partial(_deg_body, npad, n_vecs),
        out_type=jax.ShapeDtypeStruct((NC, nrow, DLANES), jnp.float32),
        mesh=mesh,
        compiler_params=pltpu.CompilerParams(needs_layout_passes=False),
        scratch_types=[
            pltpu.VMEM((n_vecs, 16), jnp.int32),
            pltpu.VMEM((nrow, DLANES), jnp.float32),
            pltpu.VMEM((nrow,), jnp.int32),
            pltpu.VMEM_SHARED((nrow, DLANES), jnp.float32),
        ],
    )(row16, zeros_row)
    deg2d = deg.reshape(NC, npad)

    rb = 512
    grid = ((n + rb - 1) // rb,)

    y = pl.pallas_call(
        _scale_body,
        grid=grid,
        in_specs=[
            pl.BlockSpec((rb, d_in), lambda i: (i, 0)),
            pl.BlockSpec((NC, rb), lambda i: (0, i)),
        ],
        out_specs=pl.BlockSpec((rb, d_in), lambda i: (i, 0)),
        out_shape=jax.ShapeDtypeStruct((n, d_in), jnp.float32),
    )(x, deg2d)

    z = pl.kernel(
        functools.partial(_agg_body, npad, n_chunks, d_in),
        out_type=jax.ShapeDtypeStruct((NC, npad, d_in), jnp.float32),
        mesh=mesh,
        scratch_types=[
            pltpu.VMEM((KB, CH), jnp.int32),
            pltpu.VMEM((KB, CH), jnp.int32),
            pltpu.VMEM((CH, d_in), jnp.float32),
            pltpu.VMEM((CH, d_in), jnp.float32),
            pltpu.VMEM_SHARED((npad, d_in), jnp.float32),
            pltpu.SemaphoreType.DMA,
            pltpu.SemaphoreType.DMA,
            pltpu.SemaphoreType.DMA,
            pltpu.SemaphoreType.DMA,
        ],
    )(row2d, col2d, y, zeros_nd)

    out = pl.pallas_call(
        _out_body,
        grid=grid,
        in_specs=[
            pl.BlockSpec((NC, rb, d_in), lambda i: (0, i, 0)),
            pl.BlockSpec((NC, rb), lambda i: (0, i)),
            pl.BlockSpec((d_out, d_in), lambda i: (0, 0)),
            pl.BlockSpec((1, d_out), lambda i: (0, 0)),
        ],
        out_specs=pl.BlockSpec((rb, d_out), lambda i: (i, 0)),
        out_shape=jax.ShapeDtypeStruct((n, d_out), jnp.float32),
    )(z, deg2d, W, b.reshape(1, d_out))
    return out
